# Initial kernel scaffold; baseline (speedup 1.0000x reference)
import functools
import numpy as np
import jax
import jax.numpy as jnp
from jax import lax
from jax.experimental import pallas as pl
from jax.experimental.pallas import tpu as pltpu
from jax.experimental.pallas import tpu_sc as plsc

_N = 10000
_E = 160000
_MUL = 16
_NB = 10
_H = 256
_WN = 512

_SCALE = float(np.sqrt(2.0) / 256.0)
_INV_SQRT10 = float(1.0 / np.sqrt(10.0))

_EB = 1280
_GRID = _E // _EB

_ROWS = _E // 128
_RPW = _ROWS // 32
_RPT = _ROWS // 16
_STRIPE = _N // 16


def _build_consts():
    col = np.arange(_WN)
    i_of_col = (col % 256) // 16
    t = (np.arange(_MUL)[:, None] == i_of_col[None, :]).astype(np.float32)
    r = np.zeros((_WN, 64), np.float32)
    q = np.zeros((4, 64), np.float32)
    q[0, :16] = _SCALE
    for i in range(16):
        for k in range(16):
            r[i * 16 + k, k] = 1.0
            for d in range(3):
                r[256 + i * 16 + k, 16 + k * 3 + d] = 1.0
    for k in range(16):
        for d in range(3):
            q[1 + d, 16 + k * 3 + d] = _SCALE
    return jnp.asarray(t), jnp.asarray(r), jnp.asarray(q)


def _tc_body(es_ref, x_ref, sh_ref, w1_ref, w2_ref, t_ref, r_ref, q_ref,
             out_ref):
    es = es_ref[...]
    h = jnp.dot(es, w1_ref[...], preferred_element_type=jnp.float32)
    h = jnp.maximum(h * _INV_SQRT10, 0.0)
    w = jnp.dot(h, w2_ref[...], preferred_element_type=jnp.float32)
    xb = jnp.dot(x_ref[...], t_ref[...], preferred_element_type=jnp.float32)
    m = w * xb
    y = jnp.dot(m, r_ref[...], preferred_element_type=jnp.float32)
    shb = jnp.dot(sh_ref[...], q_ref[...], preferred_element_type=jnp.float32)
    o = y * shb
    out_ref[0] = o[:, :32]
    out_ref[1] = o[:, 32:]


def _tc_call(edge_scalars, x_src, edge_sh, w1, w2):
    t, r, q = _build_consts()
    return pl.pallas_call(
        _tc_body,
        grid=(_GRID,),
        in_specs=[
            pl.BlockSpec((_EB, _NB), lambda g: (g, 0)),
            pl.BlockSpec((_EB, _MUL), lambda g: (g, 0)),
            pl.BlockSpec((_EB, 4), lambda g: (g, 0)),
            pl.BlockSpec((_NB, _H), lambda g: (0, 0)),
            pl.BlockSpec((_H, _WN), lambda g: (0, 0)),
            pl.BlockSpec((_MUL, _WN), lambda g: (0, 0)),
            pl.BlockSpec((_WN, 64), lambda g: (0, 0)),
            pl.BlockSpec((4, 64), lambda g: (0, 0)),
        ],
        out_specs=pl.BlockSpec((2, _EB, 32), lambda g: (0, g, 0)),
        out_shape=jax.ShapeDtypeStruct((2, _E, 32), jnp.float32),
    )(edge_scalars, x_src, edge_sh, w1, w2, t, r, q)


def _gather_call(nf, src2):
    mesh = plsc.VectorSubcoreMesh(core_axis_name="c", subcore_axis_name="s")

    @functools.partial(
        pl.kernel,
        out_type=jax.ShapeDtypeStruct((_E, _MUL), jnp.float32),
        mesh=mesh,
        scratch_types=[
            pltpu.VMEM((_RPW, 128), jnp.int32),
            pltpu.VMEM((_RPW * 128, _MUL), jnp.float32),
            pltpu.SemaphoreType.DMA,
        ],
    )
    def gather_k(nf_hbm, src_hbm, out_hbm, idx_b, rows_b, sem):
        wid = lax.axis_index("s") * 2 + lax.axis_index("c")
        r0 = wid * _RPW
        pltpu.sync_copy(src_hbm.at[pl.ds(r0, _RPW)], idx_b)

        def chunk(k, carry):
            base = k * 13
            handles = []
            for j in range(13):
                row = base + j
                handles.append(pltpu.async_copy(
                    nf_hbm.at[idx_b.at[row]],
                    rows_b.at[pl.ds(row * 128, 128)], sem))
            for h in handles:
                h.wait()
            return carry

        lax.fori_loop(0, 3, chunk, 0)
        pltpu.sync_copy(rows_b, out_hbm.at[pl.ds(wid * (_RPW * 128),
                                                 _RPW * 128)])

        @pl.when(wid < 2)
        def _():
            pltpu.sync_copy(src_hbm.at[pl.ds(32 * _RPW + wid, 1)],
                            idx_b.at[pl.ds(0, 1)])
            pltpu.async_copy(nf_hbm.at[idx_b.at[0]],
                             rows_b.at[pl.ds(0, 128)], sem).wait()
            pltpu.sync_copy(rows_b.at[pl.ds(0, 128)],
                            out_hbm.at[pl.ds((32 * _RPW + wid) * 128, 128)])

    return gather_k(nf, src2)


def _scatter_call(ef2, dst2):
    mesh = plsc.VectorSubcoreMesh(core_axis_name="c", subcore_axis_name="s")

    @functools.partial(
        pl.kernel,
        out_type=jax.ShapeDtypeStruct((2, _N, 32), jnp.float32),
        mesh=mesh,
        scratch_types=[
            pltpu.VMEM((6, 128), jnp.int32),
            pltpu.VMEM((768, 32), jnp.float32),
            pltpu.VMEM((_STRIPE, 32), jnp.float32),
            pltpu.VMEM_SHARED((_N, 32), jnp.float32),
        ],
    )
    def scatter_k(ef_hbm, dst_hbm, out_hbm, idx_b, val_b, zb, accum):
        c = lax.axis_index("c")
        s = lax.axis_index("s")
        zeros16 = jnp.zeros((16,), jnp.float32)

        def zrow(rr, carry):
            zb[rr, pl.ds(0, 16)] = zeros16
            zb[rr, pl.ds(16, 16)] = zeros16
            return carry

        lax.fori_loop(0, _STRIPE, zrow, 0)
        pltpu.sync_copy(zb, accum.at[pl.ds(s * _STRIPE, _STRIPE)])
        plsc.subcore_barrier()

        def chunk(k, carry):
            r0 = s * _RPT + k * 6
            pltpu.sync_copy(dst_hbm.at[pl.ds(r0, 6)], idx_b)
            pltpu.sync_copy(ef_hbm.at[c, pl.ds(r0 * 128, 768)], val_b)
            for j in range(6):
                pltpu.sync_copy(val_b.at[pl.ds(j * 128, 128)],
                                accum.at[idx_b.at[j]], add=True)
            return carry

        lax.fori_loop(0, 13, chunk, 0)

        @pl.when(s < 2)
        def _():
            row = 16 * _RPT + s
            pltpu.sync_copy(dst_hbm.at[pl.ds(row, 1)], idx_b.at[pl.ds(0, 1)])
            pltpu.sync_copy(ef_hbm.at[c, pl.ds(row * 128, 128)],
                            val_b.at[pl.ds(0, 128)])
            pltpu.sync_copy(val_b.at[pl.ds(0, 128)],
                            accum.at[idx_b.at[0]], add=True)

        plsc.subcore_barrier()
        pltpu.sync_copy(accum.at[pl.ds(s * _STRIPE, _STRIPE)], zb)
        pltpu.sync_copy(zb, out_hbm.at[c, pl.ds(s * _STRIPE, _STRIPE)])

    return scatter_k(ef2, dst2)


def kernel(node_features, edge_src, edge_dst, edge_sh, edge_scalars, W1, W2):
    src2 = edge_src.astype(jnp.int32).reshape(_ROWS, 128)
    dst2 = edge_dst.astype(jnp.int32).reshape(_ROWS, 128)
    x_src = _gather_call(node_features, src2)
    ef2 = _tc_call(edge_scalars, x_src, edge_sh, W1, W2)
    parts = _scatter_call(ef2, dst2)
    return jnp.concatenate([parts[0], parts[1]], axis=1)


# speedup vs baseline: 1.8809x; 1.8809x over previous
import functools
import numpy as np
import jax
import jax.numpy as jnp
from jax import lax
from jax.experimental import pallas as pl
from jax.experimental.pallas import tpu as pltpu
from jax.experimental.pallas import tpu_sc as plsc

_N = 10000
_E = 160000
_MUL = 16
_NB = 10
_H = 256
_WN = 512

_EP = 163840
_NP = 10240

_SCALE = float(np.sqrt(2.0) / 256.0)
_INV_SQRT10 = float(1.0 / np.sqrt(10.0))

_EB = 1280
_GRID = _EP // _EB

_ROWS = _EP // 128
_RPW = _ROWS // 32
_RPT = _ROWS // 16
_STRIPE = _NP // 16


def _build_consts():
    col = np.arange(_WN)
    i_of_col = (col % 256) // 16
    t = (np.arange(_MUL)[:, None] == i_of_col[None, :]).astype(np.float32)
    r = np.zeros((_WN, 64), np.float32)
    q = np.zeros((4, 64), np.float32)
    q[0, :16] = _SCALE
    for i in range(16):
        for k in range(16):
            r[i * 16 + k, k] = 1.0
            for d in range(3):
                r[256 + i * 16 + k, 16 + k * 3 + d] = 1.0
    for k in range(16):
        for d in range(3):
            q[1 + d, 16 + k * 3 + d] = _SCALE
    return jnp.asarray(t), jnp.asarray(r), jnp.asarray(q)


def _tc_body(es_ref, x_ref, sh_ref, w1_ref, w2_ref, t_ref, r_ref, q_ref,
             out_ref):
    es = es_ref[...]
    h = jnp.dot(es, w1_ref[...], preferred_element_type=jnp.float32)
    h = jnp.maximum(h * _INV_SQRT10, 0.0)
    w = jnp.dot(h, w2_ref[...], preferred_element_type=jnp.float32)
    xb = jnp.dot(x_ref[...], t_ref[...], preferred_element_type=jnp.float32)
    m = w * xb
    y = jnp.dot(m, r_ref[...], preferred_element_type=jnp.float32)
    shb = jnp.dot(sh_ref[...], q_ref[...], preferred_element_type=jnp.float32)
    o = y * shb
    out_ref[0] = o[:, :32]
    out_ref[1] = o[:, 32:]


def _tc_call(edge_scalars, x_src, edge_sh, w1, w2):
    t, r, q = _build_consts()
    return pl.pallas_call(
        _tc_body,
        grid=(_GRID,),
        in_specs=[
            pl.BlockSpec((_EB, _NB), lambda g: (g, 0)),
            pl.BlockSpec((_EB, _MUL), lambda g: (g, 0)),
            pl.BlockSpec((_EB, 4), lambda g: (g, 0)),
            pl.BlockSpec((_NB, _H), lambda g: (0, 0)),
            pl.BlockSpec((_H, _WN), lambda g: (0, 0)),
            pl.BlockSpec((_MUL, _WN), lambda g: (0, 0)),
            pl.BlockSpec((_WN, 64), lambda g: (0, 0)),
            pl.BlockSpec((4, 64), lambda g: (0, 0)),
        ],
        out_specs=pl.BlockSpec((2, _EB, 32), lambda g: (0, g, 0)),
        out_shape=jax.ShapeDtypeStruct((2, _EP, 32), jnp.float32),
    )(edge_scalars, x_src, edge_sh, w1, w2, t, r, q)


def _gather_call(nf, src2):
    mesh = plsc.VectorSubcoreMesh(core_axis_name="c", subcore_axis_name="s")

    @functools.partial(
        pl.kernel,
        out_type=jax.ShapeDtypeStruct((_EP, _MUL), jnp.float32),
        mesh=mesh,
        scratch_types=[
            pltpu.VMEM((_RPW, 128), jnp.int32),
            pltpu.VMEM((_RPW * 128, _MUL), jnp.float32),
            pltpu.SemaphoreType.DMA,
        ],
        compiler_params=pltpu.CompilerParams(use_tc_tiling_on_sc=False),
    )
    def gather_k(nf_hbm, src_hbm, out_hbm, idx_b, rows_b, sem):
        wid = lax.axis_index("s") * 2 + lax.axis_index("c")
        r0 = wid * _RPW
        pltpu.sync_copy(src_hbm.at[pl.ds(r0, _RPW)], idx_b)

        def chunk(k, carry):
            base = k * 8
            handles = []
            for j in range(8):
                row = base + j
                handles.append(pltpu.async_copy(
                    nf_hbm.at[idx_b.at[row]],
                    rows_b.at[pl.ds(row * 128, 128)], sem))
            for h in handles:
                h.wait()
            return carry

        lax.fori_loop(0, _RPW // 8, chunk, 0)
        pltpu.sync_copy(rows_b, out_hbm.at[pl.ds(wid * (_RPW * 128),
                                                 _RPW * 128)])

    return gather_k(nf, src2)


def _scatter_call(ef2, dst2):
    mesh = plsc.VectorSubcoreMesh(core_axis_name="c", subcore_axis_name="s")

    @functools.partial(
        pl.kernel,
        out_type=jax.ShapeDtypeStruct((2, _NP, 32), jnp.float32),
        mesh=mesh,
        scratch_types=[
            pltpu.VMEM((8, 128), jnp.int32),
            pltpu.VMEM((1024, 32), jnp.float32),
            pltpu.VMEM((_STRIPE, 32), jnp.float32),
            pltpu.VMEM_SHARED((_NP, 32), jnp.float32),
        ],
        compiler_params=pltpu.CompilerParams(use_tc_tiling_on_sc=False),
    )
    def scatter_k(ef_hbm, dst_hbm, out_hbm, idx_b, val_b, zb, accum):
        c = lax.axis_index("c")
        s = lax.axis_index("s")
        zeros16 = jnp.zeros((16,), jnp.float32)

        def zrow(rr, carry):
            zb[rr, pl.ds(0, 16)] = zeros16
            zb[rr, pl.ds(16, 16)] = zeros16
            return carry

        lax.fori_loop(0, _STRIPE, zrow, 0)
        pltpu.sync_copy(zb, accum.at[pl.ds(s * _STRIPE, _STRIPE)])
        plsc.subcore_barrier()

        def chunk(k, carry):
            r0 = s * _RPT + k * 8
            pltpu.sync_copy(dst_hbm.at[pl.ds(r0, 8)], idx_b)
            pltpu.sync_copy(ef_hbm.at[c, pl.ds(r0 * 128, 1024)], val_b)
            for j in range(8):
                pltpu.sync_copy(val_b.at[pl.ds(j * 128, 128)],
                                accum.at[idx_b.at[j]], add=True)
            return carry

        lax.fori_loop(0, _RPT // 8, chunk, 0)
        plsc.subcore_barrier()
        pltpu.sync_copy(accum.at[pl.ds(s * _STRIPE, _STRIPE)], zb)
        pltpu.sync_copy(zb, out_hbm.at[c, pl.ds(s * _STRIPE, _STRIPE)])

    return scatter_k(ef2, dst2)


def kernel(node_features, edge_src, edge_dst, edge_sh, edge_scalars, W1, W2):
    pad = _EP - _E
    src2 = jnp.pad(edge_src.astype(jnp.int32), (0, pad)).reshape(_ROWS, 128)
    dst2 = jnp.pad(edge_dst.astype(jnp.int32), (0, pad)).reshape(_ROWS, 128)
    es_p = jnp.pad(edge_scalars, ((0, pad), (0, 0)))
    sh_p = jnp.pad(edge_sh, ((0, pad), (0, 0)))
    x_src = _gather_call(node_features, src2)
    ef2 = _tc_call(es_p, x_src, sh_p, W1, W2)
    parts = _scatter_call(ef2, dst2)
    return jnp.concatenate([parts[0], parts[1]], axis=1)[:_N]


# speedup vs baseline: 2.1674x; 1.1524x over previous
import functools
import numpy as np
import jax
import jax.numpy as jnp
from jax import lax
from jax.experimental import pallas as pl
from jax.experimental.pallas import tpu as pltpu
from jax.experimental.pallas import tpu_sc as plsc

_N = 10000
_E = 160000
_MUL = 16
_NB = 10
_H = 256
_WN = 512

_SCALE = float(np.sqrt(2.0) / 256.0)
_INV_SQRT10 = float(1.0 / np.sqrt(10.0))

_EB = 1280
_GRID = _E // _EB

_ROWS = _E // 128
_RPW = _ROWS // 32
_RPT = _ROWS // 16
_STRIPE = _N // 16


def _build_consts():
    col = np.arange(_WN)
    i_of_col = (col % 256) // 16
    t = (np.arange(_MUL)[:, None] == i_of_col[None, :]).astype(np.float32)
    r = np.zeros((_WN, 64), np.float32)
    q = np.zeros((4, 64), np.float32)
    q[0, :16] = _SCALE
    for i in range(16):
        for k in range(16):
            r[i * 16 + k, k] = 1.0
            for d in range(3):
                r[256 + i * 16 + k, 16 + k * 3 + d] = 1.0
    for k in range(16):
        for d in range(3):
            q[1 + d, 16 + k * 3 + d] = _SCALE
    return jnp.asarray(t), jnp.asarray(r), jnp.asarray(q)


def _tc_body(es_ref, x_ref, sh_ref, w1_ref, w2_ref, t_ref, r_ref, q_ref,
             out_ref):
    es = es_ref[...]
    h = jnp.dot(es, w1_ref[...], preferred_element_type=jnp.float32)
    h = jnp.maximum(h * _INV_SQRT10, 0.0)
    w = jnp.dot(h, w2_ref[...], preferred_element_type=jnp.float32)
    xb = jnp.dot(x_ref[...], t_ref[...], preferred_element_type=jnp.float32)
    m = w * xb
    y = jnp.dot(m, r_ref[...], preferred_element_type=jnp.float32)
    shb = jnp.dot(sh_ref[...], q_ref[...], preferred_element_type=jnp.float32)
    o = y * shb
    out_ref[0] = o[:, :32]
    out_ref[1] = o[:, 32:]


def _tc_call(edge_scalars, x_src, edge_sh, w1, w2):
    t, r, q = _build_consts()
    return pl.pallas_call(
        _tc_body,
        grid=(_GRID,),
        in_specs=[
            pl.BlockSpec((_EB, _NB), lambda g: (g, 0)),
            pl.BlockSpec((_EB, _MUL), lambda g: (g, 0)),
            pl.BlockSpec((_EB, 4), lambda g: (g, 0)),
            pl.BlockSpec((_NB, _H), lambda g: (0, 0)),
            pl.BlockSpec((_H, _WN), lambda g: (0, 0)),
            pl.BlockSpec((_MUL, _WN), lambda g: (0, 0)),
            pl.BlockSpec((_WN, 64), lambda g: (0, 0)),
            pl.BlockSpec((4, 64), lambda g: (0, 0)),
        ],
        out_specs=pl.BlockSpec((2, _EB, 32), lambda g: (0, g, 0)),
        out_shape=jax.ShapeDtypeStruct((2, _E, 32), jnp.float32),
    )(edge_scalars, x_src, edge_sh, w1, w2, t, r, q)


def _gather_call(nf, src2):
    mesh = plsc.VectorSubcoreMesh(core_axis_name="c", subcore_axis_name="s")

    @functools.partial(
        pl.kernel,
        out_type=jax.ShapeDtypeStruct((_E, _MUL), jnp.float32),
        mesh=mesh,
        scratch_types=[
            pltpu.VMEM((_RPW + 1, 128), jnp.int32),
            pltpu.VMEM(((_RPW + 1) * 128, _MUL), jnp.float32),
            pltpu.SemaphoreType.DMA,
        ],
        compiler_params=pltpu.CompilerParams(use_tc_tiling_on_sc=False),
    )
    def gather_k(nf_hbm, src_hbm, out_hbm, idx_b, rows_b, sem):
        wid = lax.axis_index("s") * 2 + lax.axis_index("c")
        r0 = wid * _RPW
        pltpu.sync_copy(src_hbm.at[pl.ds(r0, _RPW)],
                        idx_b.at[pl.ds(0, _RPW)])

        def chunk(k, carry):
            base = k * 13
            handles = []
            for j in range(13):
                row = base + j
                handles.append(pltpu.async_copy(
                    nf_hbm.at[idx_b.at[row]],
                    rows_b.at[pl.ds(row * 128, 128)], sem))
            for h in handles:
                h.wait()
            return carry

        lax.fori_loop(0, _RPW // 13, chunk, 0)
        pltpu.sync_copy(rows_b.at[pl.ds(0, _RPW * 128)],
                        out_hbm.at[pl.ds(wid * (_RPW * 128), _RPW * 128)])

        @pl.when(wid < 2)
        def _():
            row = 32 * _RPW + wid
            pltpu.sync_copy(src_hbm.at[pl.ds(row, 1)],
                            idx_b.at[pl.ds(_RPW, 1)])
            pltpu.async_copy(nf_hbm.at[idx_b.at[_RPW]],
                             rows_b.at[pl.ds(_RPW * 128, 128)], sem).wait()
            pltpu.sync_copy(rows_b.at[pl.ds(_RPW * 128, 128)],
                            out_hbm.at[pl.ds(row * 128, 128)])

    return gather_k(nf, src2)


def _scatter_call(ef2, dst2):
    mesh = plsc.VectorSubcoreMesh(core_axis_name="c", subcore_axis_name="s")

    @functools.partial(
        pl.kernel,
        out_type=jax.ShapeDtypeStruct((2, _N, 32), jnp.float32),
        mesh=mesh,
        scratch_types=[
            pltpu.VMEM((6, 128), jnp.int32),
            pltpu.VMEM((768, 32), jnp.float32),
            pltpu.VMEM((_STRIPE, 32), jnp.float32),
            pltpu.VMEM_SHARED((_N, 32), jnp.float32),
        ],
        compiler_params=pltpu.CompilerParams(use_tc_tiling_on_sc=False),
    )
    def scatter_k(ef_hbm, dst_hbm, out_hbm, idx_b, val_b, zb, accum):
        c = lax.axis_index("c")
        s = lax.axis_index("s")
        zeros16 = jnp.zeros((16,), jnp.float32)

        def zrow(rr, carry):
            zb[rr, pl.ds(0, 16)] = zeros16
            zb[rr, pl.ds(16, 16)] = zeros16
            return carry

        lax.fori_loop(0, _STRIPE, zrow, 0)
        pltpu.sync_copy(zb, accum.at[pl.ds(s * _STRIPE, _STRIPE)])
        plsc.subcore_barrier()

        def chunk(k, carry):
            r0 = s * _RPT + k * 6
            pltpu.sync_copy(dst_hbm.at[pl.ds(r0, 6)], idx_b)
            pltpu.sync_copy(ef_hbm.at[c, pl.ds(r0 * 128, 768)], val_b)
            for j in range(6):
                pltpu.sync_copy(val_b.at[pl.ds(j * 128, 128)],
                                accum.at[idx_b.at[j]], add=True)
            return carry

        lax.fori_loop(0, _RPT // 6, chunk, 0)

        @pl.when(s < 2)
        def _():
            row = 16 * _RPT + s
            pltpu.sync_copy(dst_hbm.at[pl.ds(row, 1)], idx_b.at[pl.ds(0, 1)])
            pltpu.sync_copy(ef_hbm.at[c, pl.ds(row * 128, 128)],
                            val_b.at[pl.ds(0, 128)])
            pltpu.sync_copy(val_b.at[pl.ds(0, 128)],
                            accum.at[idx_b.at[0]], add=True)

        plsc.subcore_barrier()
        pltpu.sync_copy(accum.at[pl.ds(s * _STRIPE, _STRIPE)], zb)
        pltpu.sync_copy(zb, out_hbm.at[c, pl.ds(s * _STRIPE, _STRIPE)])

    return scatter_k(ef2, dst2)


def kernel(node_features, edge_src, edge_dst, edge_sh, edge_scalars, W1, W2):
    src2 = edge_src.astype(jnp.int32).reshape(_ROWS, 128)
    dst2 = edge_dst.astype(jnp.int32).reshape(_ROWS, 128)
    x_src = _gather_call(node_features, src2)
    ef2 = _tc_call(edge_scalars, x_src, edge_sh, W1, W2)
    parts = _scatter_call(ef2, dst2)
    return jnp.concatenate([parts[0], parts[1]], axis=1)


# speedup vs baseline: 2.4312x; 1.1217x over previous
import functools
import numpy as np
import jax
import jax.numpy as jnp
from jax import lax
from jax.experimental import pallas as pl
from jax.experimental.pallas import tpu as pltpu
from jax.experimental.pallas import tpu_sc as plsc

_N = 10000
_E = 160000
_MUL = 16
_NB = 10
_H = 256
_WN = 512

_SCALE = float(np.sqrt(2.0) / 256.0)
_INV_SQRT10 = float(1.0 / np.sqrt(10.0))

_EB = 1280
_GRID = _E // _EB

_ROWS = _E // 128
_RPW = _ROWS // 32
_RPT = _ROWS // 16
_STRIPE = _N // 16


def _build_consts():
    col = np.arange(_WN)
    i_of_col = (col % 256) // 16
    t = (np.arange(_MUL)[:, None] == i_of_col[None, :]).astype(np.float32)
    r = np.zeros((_WN, 64), np.float32)
    q = np.zeros((4, 64), np.float32)
    q[0, :16] = _SCALE
    for i in range(16):
        for k in range(16):
            r[i * 16 + k, k] = 1.0
            for d in range(3):
                r[256 + i * 16 + k, 16 + k * 3 + d] = 1.0
    for k in range(16):
        for d in range(3):
            q[1 + d, 16 + k * 3 + d] = _SCALE
    return jnp.asarray(t), jnp.asarray(r), jnp.asarray(q)


def _tc_body(es_ref, x_ref, sh_ref, w1_ref, w2_ref, t_ref, r_ref, q_ref,
             out_ref):
    es = es_ref[...]
    h = jnp.dot(es, w1_ref[...], preferred_element_type=jnp.float32)
    h = jnp.maximum(h, 0.0).astype(jnp.bfloat16)
    w = jnp.dot(h, w2_ref[...],
                preferred_element_type=jnp.float32).astype(jnp.bfloat16)
    xb = jnp.dot(x_ref[...].astype(jnp.bfloat16), t_ref[...],
                 preferred_element_type=jnp.float32).astype(jnp.bfloat16)
    m = w * xb
    y = jnp.dot(m, r_ref[...], preferred_element_type=jnp.float32)
    shb = jnp.dot(sh_ref[...], q_ref[...], preferred_element_type=jnp.float32)
    out_ref[...] = y * shb


def _tc_call(edge_scalars, x_src, edge_sh, w1, w2):
    t, r, q = _build_consts()
    return pl.pallas_call(
        _tc_body,
        grid=(_GRID,),
        in_specs=[
            pl.BlockSpec((_EB, _NB), lambda g: (g, 0)),
            pl.BlockSpec((_EB, _MUL), lambda g: (g, 0)),
            pl.BlockSpec((_EB, 4), lambda g: (g, 0)),
            pl.BlockSpec((_NB, _H), lambda g: (0, 0)),
            pl.BlockSpec((_H, _WN), lambda g: (0, 0)),
            pl.BlockSpec((_MUL, _WN), lambda g: (0, 0)),
            pl.BlockSpec((_WN, 64), lambda g: (0, 0)),
            pl.BlockSpec((4, 64), lambda g: (0, 0)),
        ],
        out_specs=pl.BlockSpec((_EB, 64), lambda g: (g, 0)),
        out_shape=jax.ShapeDtypeStruct((_E, 64), jnp.float32),
    )(edge_scalars, x_src, edge_sh, w1 * _INV_SQRT10,
      w2.astype(jnp.bfloat16), t.astype(jnp.bfloat16),
      r.astype(jnp.bfloat16), q)


def _gather_call(nf, src2):
    mesh = plsc.VectorSubcoreMesh(core_axis_name="c", subcore_axis_name="s")

    @functools.partial(
        pl.kernel,
        out_type=jax.ShapeDtypeStruct((_E, _MUL), jnp.float32),
        mesh=mesh,
        scratch_types=[
            pltpu.VMEM((_RPW + 1, 128), jnp.int32),
            pltpu.VMEM(((_RPW + 1) * 128, _MUL), jnp.float32),
            pltpu.SemaphoreType.DMA,
        ],
        compiler_params=pltpu.CompilerParams(use_tc_tiling_on_sc=False),
    )
    def gather_k(nf_hbm, src_hbm, out_hbm, idx_b, rows_b, sem):
        wid = lax.axis_index("s") * 2 + lax.axis_index("c")
        r0 = wid * _RPW
        pltpu.sync_copy(src_hbm.at[pl.ds(r0, _RPW)],
                        idx_b.at[pl.ds(0, _RPW)])

        def chunk(k, carry):
            base = k * 13
            handles = []
            for j in range(13):
                row = base + j
                handles.append(pltpu.async_copy(
                    nf_hbm.at[idx_b.at[row]],
                    rows_b.at[pl.ds(row * 128, 128)], sem))
            for h in handles:
                h.wait()
            return carry

        lax.fori_loop(0, _RPW // 13, chunk, 0)
        pltpu.sync_copy(rows_b.at[pl.ds(0, _RPW * 128)],
                        out_hbm.at[pl.ds(wid * (_RPW * 128), _RPW * 128)])

        @pl.when(wid < 2)
        def _():
            row = 32 * _RPW + wid
            pltpu.sync_copy(src_hbm.at[pl.ds(row, 1)],
                            idx_b.at[pl.ds(_RPW, 1)])
            pltpu.async_copy(nf_hbm.at[idx_b.at[_RPW]],
                             rows_b.at[pl.ds(_RPW * 128, 128)], sem).wait()
            pltpu.sync_copy(rows_b.at[pl.ds(_RPW * 128, 128)],
                            out_hbm.at[pl.ds(row * 128, 128)])

    return gather_k(nf, src2)


def _scatter_call(ef2, dst2):
    mesh = plsc.VectorSubcoreMesh(core_axis_name="c", subcore_axis_name="s")

    @functools.partial(
        pl.kernel,
        out_type=jax.ShapeDtypeStruct((2, _N, 32), jnp.float32),
        mesh=mesh,
        scratch_types=[
            pltpu.VMEM((6, 128), jnp.int32),
            pltpu.VMEM((768, 32), jnp.float32),
            pltpu.VMEM((_STRIPE, 32), jnp.float32),
            pltpu.VMEM_SHARED((_N, 32), jnp.float32),
        ],
        compiler_params=pltpu.CompilerParams(use_tc_tiling_on_sc=False),
    )
    def scatter_k(ef_hbm, dst_hbm, out_hbm, idx_b, val_b, zb, accum):
        c = lax.axis_index("c")
        s = lax.axis_index("s")
        zeros16 = jnp.zeros((16,), jnp.float32)

        def zrow(rr, carry):
            zb[rr, pl.ds(0, 16)] = zeros16
            zb[rr, pl.ds(16, 16)] = zeros16
            return carry

        lax.fori_loop(0, _STRIPE, zrow, 0)
        pltpu.sync_copy(zb, accum.at[pl.ds(s * _STRIPE, _STRIPE)])
        plsc.subcore_barrier()

        def chunk(k, carry):
            r0 = s * _RPT + k * 6
            pltpu.sync_copy(dst_hbm.at[pl.ds(r0, 6)], idx_b)
            pltpu.sync_copy(ef_hbm.at[pl.ds(r0 * 128, 768),
                                      pl.ds(c * 32, 32)], val_b)
            for j in range(6):
                pltpu.sync_copy(val_b.at[pl.ds(j * 128, 128)],
                                accum.at[idx_b.at[j]], add=True)
            return carry

        lax.fori_loop(0, _RPT // 6, chunk, 0)

        @pl.when(s < 2)
        def _():
            row = 16 * _RPT + s
            pltpu.sync_copy(dst_hbm.at[pl.ds(row, 1)], idx_b.at[pl.ds(0, 1)])
            pltpu.sync_copy(ef_hbm.at[pl.ds(row * 128, 128),
                                      pl.ds(c * 32, 32)],
                            val_b.at[pl.ds(0, 128)])
            pltpu.sync_copy(val_b.at[pl.ds(0, 128)],
                            accum.at[idx_b.at[0]], add=True)

        plsc.subcore_barrier()
        pltpu.sync_copy(accum.at[pl.ds(s * _STRIPE, _STRIPE)], zb)
        pltpu.sync_copy(zb, out_hbm.at[c, pl.ds(s * _STRIPE, _STRIPE)])

    return scatter_k(ef2, dst2)


def kernel(node_features, edge_src, edge_dst, edge_sh, edge_scalars, W1, W2):
    src2 = edge_src.astype(jnp.int32).reshape(_ROWS, 128)
    dst2 = edge_dst.astype(jnp.int32).reshape(_ROWS, 128)
    x_src = _gather_call(node_features, src2)
    ef2 = _tc_call(edge_scalars, x_src, edge_sh, W1, W2)
    parts = _scatter_call(ef2, dst2)
    return jnp.concatenate([parts[0], parts[1]], axis=1)
